# baseline (speedup 1.0000x reference)
import functools

import jax
import jax.numpy as jnp
from jax import lax
from jax.experimental import pallas as pl
from jax.experimental.pallas import tpu as pltpu
from jax.experimental.pallas import tpu_sc as plsc

NUM_EMB = 1_000_000
DIM = 32
BATCH = 16384
SEQ = 50
SEQ_P = 56
LANE_P = 128
NC, NS = 2, 16
NW = NC * NS
ROWS_PER_W = BATCH // NW
BB = 16
NCHUNKS = ROWS_PER_W // BB
assert ROWS_PER_W % BB == 0 and NCHUNKS >= 2


def _emb_body(ids_hbm, table_hbm, out_hbm, idx_v, rows_v, sem_idx, sem_g, sem_out):
    wid = lax.axis_index("s") * NC + lax.axis_index("c")
    base = wid * ROWS_PER_W

    def idx_copy(c, b):
        return pltpu.make_async_copy(
            ids_hbm.at[pl.ds(base + c * BB, BB), :], idx_v.at[b], sem_idx
        )

    def out_copy(c, b):
        return pltpu.make_async_copy(
            rows_v.at[b],
            out_hbm.at[pl.ds(base + c * BB, BB), :, :],
            sem_out,
        )

    idx_copy(0, 0).start()

    @pl.loop(0, NCHUNKS)
    def _chunk(c):
        b = lax.rem(c, 2)
        idx_copy(c, b).wait()
        @pl.when(c >= 2)
        def _():
            out_copy(c - 2, b).wait()

        gathers = [
            pltpu.make_async_copy(
                table_hbm.at[idx_v.at[b].at[i, pl.ds(0, SEQ_P)]],
                rows_v.at[b].at[i],
                sem_g,
            )
            for i in range(BB)
        ]
        for g in gathers:
            g.start()

        @pl.when(c + 1 < NCHUNKS)
        def _():
            idx_copy(c + 1, 1 - b).start()

        for g in gathers:
            g.wait()
        out_copy(c, b).start()

    out_copy(NCHUNKS - 2, (NCHUNKS - 2) % 2).wait()
    out_copy(NCHUNKS - 1, (NCHUNKS - 1) % 2).wait()


_emb_call = functools.partial(
    pl.kernel,
    out_type=jax.ShapeDtypeStruct((BATCH, SEQ_P, DIM), jnp.float32),
    mesh=plsc.VectorSubcoreMesh(
        core_axis_name="c", subcore_axis_name="s", num_cores=NC, num_subcores=NS
    ),
    scratch_types=[
        pltpu.VMEM((2, BB, LANE_P), jnp.int32),
        pltpu.VMEM((2, BB, SEQ_P, DIM), jnp.float32),
        pltpu.SemaphoreType.DMA,
        pltpu.SemaphoreType.DMA,
        pltpu.SemaphoreType.DMA,
    ],
    compiler_params=pltpu.CompilerParams(use_tc_tiling_on_sc=False),
)(_emb_body)


@jax.jit
def kernel(ids, table):
    ids_p = jnp.pad(ids.astype(jnp.int32), ((0, 0), (0, LANE_P - SEQ)))
    out_p = _emb_call(ids_p, table)
    return out_p[:, :SEQ, :]


# speedup vs baseline: 2.0675x; 2.0675x over previous
import functools

import jax
import jax.numpy as jnp
from jax import lax
from jax.experimental import pallas as pl
from jax.experimental.pallas import tpu as pltpu
from jax.experimental.pallas import tpu_sc as plsc

NUM_EMB = 1_000_000
DIM = 32
BATCH = 16384
SEQ = 50
NC, NS = 2, 16
NW = NC * NS
ROWS_PER_W = BATCH // NW
BB = 16
NCHUNKS = ROWS_PER_W // BB
assert ROWS_PER_W % BB == 0 and NCHUNKS >= 2


def _emb_body(ids_hbm, table_hbm, out_hbm, idx_v, rows_v, sem_idx, sem_g, sem_out):
    wid = lax.axis_index("s") * NC + lax.axis_index("c")
    base = wid * ROWS_PER_W

    def idx_copy(c, b):
        return pltpu.make_async_copy(
            ids_hbm.at[pl.ds(base + c * BB, BB), :], idx_v.at[b], sem_idx
        )

    def out_copy(c, b):
        return pltpu.make_async_copy(
            rows_v.at[b], out_hbm.at[pl.ds(base + c * BB, BB), :, :], sem_out
        )

    idx_copy(0, 0).start()

    @pl.loop(0, NCHUNKS)
    def _chunk(c):
        b = lax.rem(c, 2)
        idx_copy(c, b).wait()
        @pl.when(c >= 2)
        def _():
            out_copy(c - 2, b).wait()

        gathers = [
            pltpu.make_async_copy(
                table_hbm.at[idx_v.at[b].at[i]],
                rows_v.at[b].at[i],
                sem_g,
            )
            for i in range(BB)
        ]
        for g in gathers:
            g.start()

        @pl.when(c + 1 < NCHUNKS)
        def _():
            idx_copy(c + 1, 1 - b).start()

        for g in gathers:
            g.wait()
        out_copy(c, b).start()

    out_copy(NCHUNKS - 2, (NCHUNKS - 2) % 2).wait()
    out_copy(NCHUNKS - 1, (NCHUNKS - 1) % 2).wait()


_emb_call = functools.partial(
    pl.kernel,
    out_type=jax.ShapeDtypeStruct((BATCH, SEQ, DIM), jnp.float32),
    mesh=plsc.VectorSubcoreMesh(
        core_axis_name="c", subcore_axis_name="s", num_cores=NC, num_subcores=NS
    ),
    scratch_types=[
        pltpu.VMEM((2, BB, SEQ), jnp.int32),
        pltpu.VMEM((2, BB, SEQ, DIM), jnp.float32),
        pltpu.SemaphoreType.DMA,
        pltpu.SemaphoreType.DMA,
        pltpu.SemaphoreType.DMA,
    ],
    compiler_params=pltpu.CompilerParams(use_tc_tiling_on_sc=False),
)(_emb_body)


@jax.jit
def kernel(ids, table):
    return _emb_call(ids.astype(jnp.int32), table)
